# baseline (speedup 1.0000x reference)
import functools

import jax
import jax.numpy as jnp
from jax import lax
from jax.experimental import pallas as pl
from jax.experimental.pallas import tpu as pltpu
from jax.experimental.pallas import tpu_sc as plsc

B = 16384
D = 64

_info = plsc.get_sparse_core_info()
_NC, _NS, _L = _info.num_cores, _info.num_subcores, _info.num_lanes
_NW = _NC * _NS
_BPW = B // _NW
_CHUNK = 128
_NCHUNK = _BPW // _CHUNK


def _lane_take(x, idx):
    dnums = lax.GatherDimensionNumbers(
        offset_dims=(), collapsed_slice_dims=(0,), start_index_map=(0,))
    return lax.gather(x, idx[:, None], dnums, (1,),
                      mode=lax.GatherScatterMode.PROMISE_IN_BOUNDS)


def _normalize_rows(rows_ref, n_rows):
    lanes = lax.iota(jnp.int32, _L)
    perms = [lanes ^ sh for sh in (8, 4, 2, 1)]

    def body(r, carry):
        v0 = rows_ref[r, pl.ds(0, _L)]
        v1 = rows_ref[r, pl.ds(_L, _L)]
        v2 = rows_ref[r, pl.ds(2 * _L, _L)]
        v3 = rows_ref[r, pl.ds(3 * _L, _L)]
        ss = v0 * v0 + v1 * v1 + v2 * v2 + v3 * v3
        for p in perms:
            ss = ss + _lane_take(ss, p)
        s = ss
        i = lax.bitcast_convert_type(s, jnp.int32)
        y = lax.bitcast_convert_type(0x5F3759DF - (i >> 1), jnp.float32)
        nhalf = s * (-0.5)
        for _ in range(3):
            y = y * (1.5 + nhalf * y * y)
        rows_ref[r, pl.ds(0, _L)] = v0 * y
        rows_ref[r, pl.ds(_L, _L)] = v1 * y
        rows_ref[r, pl.ds(2 * _L, _L)] = v2 * y
        rows_ref[r, pl.ds(3 * _L, _L)] = v3 * y
        return carry

    lax.fori_loop(0, n_rows, body, 0)


def _sc_body(
    u_idx_hbm, p_idx_hbm, n_idx_hbm,
    user_table, item_table, user_bias, item_bias,
    out_ue, out_pe, out_ne, out_ub, out_pb, out_nb,
    idx_u, idx_p, idx_n,
    rows_u, rows_p, rows_n,
    b_u, b_p, b_n,
    sem_u, sem_p, sem_n, sem_b,
):
    wid = lax.axis_index("s") * _NC + lax.axis_index("c")
    base = wid * _BPW
    crow = wid * _NCHUNK

    pltpu.sync_copy(u_idx_hbm.at[pl.ds(crow, _NCHUNK)], idx_u)
    pltpu.sync_copy(p_idx_hbm.at[pl.ds(crow, _NCHUNK)], idx_p)
    pltpu.sync_copy(n_idx_hbm.at[pl.ds(crow, _NCHUNK)], idx_n)

    waits_u, waits_p, waits_n, waits_b = [], [], [], []
    for j in range(_NCHUNK):
        dst = pl.ds(j * _CHUNK, _CHUNK)
        waits_u.append(pltpu.async_copy(
            user_table.at[idx_u.at[j]], rows_u.at[dst], sem_u))
        waits_p.append(pltpu.async_copy(
            item_table.at[idx_p.at[j]], rows_p.at[dst], sem_p))
        waits_n.append(pltpu.async_copy(
            item_table.at[idx_n.at[j]], rows_n.at[dst], sem_n))
        waits_b.append(pltpu.async_copy(
            user_bias.at[idx_u.at[j]], b_u.at[dst], sem_b))
        waits_b.append(pltpu.async_copy(
            item_bias.at[idx_p.at[j]], b_p.at[dst], sem_b))
        waits_b.append(pltpu.async_copy(
            item_bias.at[idx_n.at[j]], b_n.at[dst], sem_b))

    for w in waits_u:
        w.wait()
    _normalize_rows(rows_u, _BPW)
    for w in waits_p:
        w.wait()
    _normalize_rows(rows_p, _BPW)
    for w in waits_n:
        w.wait()
    _normalize_rows(rows_n, _BPW)
    for w in waits_b:
        w.wait()

    obase = pl.ds(base, _BPW)
    pltpu.sync_copy(rows_u, out_ue.at[obase])
    pltpu.sync_copy(rows_p, out_pe.at[obase])
    pltpu.sync_copy(rows_n, out_ne.at[obase])
    pltpu.sync_copy(b_u, out_ub.at[obase])
    pltpu.sync_copy(b_p, out_pb.at[obase])
    pltpu.sync_copy(b_n, out_nb.at[obase])


@jax.jit
def _bpr_lookup(user2d, pos2d, neg2d, user_table, item_table,
                user_bias_table, item_bias_table):
    mesh = plsc.VectorSubcoreMesh(core_axis_name="c", subcore_axis_name="s")
    f32 = jnp.float32
    call = functools.partial(
        pl.kernel,
        mesh=mesh,
        compiler_params=pltpu.CompilerParams(use_tc_tiling_on_sc=False),
        out_type=[
            jax.ShapeDtypeStruct((B, D), f32),
            jax.ShapeDtypeStruct((B, D), f32),
            jax.ShapeDtypeStruct((B, D), f32),
            jax.ShapeDtypeStruct((B,), f32),
            jax.ShapeDtypeStruct((B,), f32),
            jax.ShapeDtypeStruct((B,), f32),
        ],
        scratch_types=[
            pltpu.VMEM((_NCHUNK, _CHUNK), jnp.int32),
            pltpu.VMEM((_NCHUNK, _CHUNK), jnp.int32),
            pltpu.VMEM((_NCHUNK, _CHUNK), jnp.int32),
            pltpu.VMEM((_BPW, D), f32),
            pltpu.VMEM((_BPW, D), f32),
            pltpu.VMEM((_BPW, D), f32),
            pltpu.VMEM((_BPW,), f32),
            pltpu.VMEM((_BPW,), f32),
            pltpu.VMEM((_BPW,), f32),
            pltpu.SemaphoreType.DMA,
            pltpu.SemaphoreType.DMA,
            pltpu.SemaphoreType.DMA,
            pltpu.SemaphoreType.DMA,
        ],
    )
    return call(_sc_body)(
        user2d, pos2d, neg2d,
        user_table, item_table, user_bias_table, item_bias_table,
    )


def kernel(user, pos_item, neg_item, user_table, item_table,
           user_bias_table, item_bias_table):
    user2d = user.reshape(B // _CHUNK, _CHUNK)
    pos2d = pos_item.reshape(B // _CHUNK, _CHUNK)
    neg2d = neg_item.reshape(B // _CHUNK, _CHUNK)
    ue, pe, ne, ub, pb, nb = _bpr_lookup(
        user2d, pos2d, neg2d, user_table, item_table,
        user_bias_table.reshape(-1), item_bias_table.reshape(-1))
    return (ue, pe, ne, ub, pb, nb)
